# baseline (speedup 1.0000x reference)
import jax
import jax.experimental.shard_map
import jax.numpy as jnp
import numpy as np
from jax.experimental import pallas as pl
from jax.experimental.pallas import tpu as pltpu

_H0 = 48
_R1 = 46 * 48
_R2 = 44 * 48
_WB = 24
_R3 = 20 * _WB
_R4 = 18 * _WB
_OFFS48 = tuple(di * 48 + dj for di in range(3) for dj in range(3))
_OFFS24 = tuple(di * _WB + dj for di in range(3) for dj in range(3))

_BF = jnp.bfloat16
_F32 = jnp.float32


_IPS = 4


def _conv_stack_kernel(p1_ref, w1_ref, w2_ref, w3_ref, w4_ref, o_ref,
                       p2, acc2, p3, p4, acc4):
    for j in range(_IPS):
        c1 = jnp.dot(p1_ref[j], w1_ref[...], preferred_element_type=_F32)
        p2[...] = jnp.maximum(c1, 0.0).astype(_BF)
        acc2[...] = jnp.dot(p2[...], w2_ref[...], preferred_element_type=_F32)

        jmax = jnp.maximum(acc2[pl.ds(0, 1056, 2), :],
                           acc2[pl.ds(1, 1056, 2), :]).reshape(22, 2, _WB, 64)
        cv2 = jnp.maximum(jnp.maximum(jmax[:, 0], jmax[:, 1]),
                          0.0).astype(_BF).reshape(528, 64)
        for t, off in enumerate(_OFFS24):
            sz = min(_R3, 528 - off)
            p3[0:sz, 64 * t:64 * (t + 1)] = cv2[off:off + sz, :]

        c3 = jnp.dot(p3[...], w3_ref[...], preferred_element_type=_F32)
        cv3 = jnp.maximum(c3, 0.0).astype(_BF)
        for t, off in enumerate(_OFFS24):
            sz = min(_R4, _R3 - off)
            p4[0:sz, 128 * t:128 * (t + 1)] = cv3[off:off + sz, :]

        acc4[...] = jnp.dot(p4[...], w4_ref[...], preferred_element_type=_F32)

        for ph in range(9):
            b0 = (2 * ph) * _WB
            b1 = b0 + _WB
            pr = jnp.maximum(
                jnp.maximum(acc4[pl.ds(b0, 12, 2), :],
                            acc4[pl.ds(b0 + 1, 12, 2), :]),
                jnp.maximum(acc4[pl.ds(b1, 12, 2), :],
                            acc4[pl.ds(b1 + 1, 12, 2), :]))
            pr = jnp.maximum(pr, 0.0)
            o_ref[j, pl.ds(ph * 9, 9), :] = pr[0:9, :]


def _conv_stack(p1, w1b, w2b, w3b, w4b):
    n = p1.shape[0]
    return pl.pallas_call(
        _conv_stack_kernel,
        out_shape=jax.ShapeDtypeStruct((n, 81, 128), _F32),
        grid=(n // _IPS,),
        in_specs=[
            pl.BlockSpec((_IPS, _R2, 25), lambda i: (i, 0, 0)),
            pl.BlockSpec((25, 288), lambda i: (0, 0)),
            pl.BlockSpec((288, 64), lambda i: (0, 0)),
            pl.BlockSpec((576, 128), lambda i: (0, 0)),
            pl.BlockSpec((1152, 128), lambda i: (0, 0)),
        ],
        out_specs=pl.BlockSpec((_IPS, 81, 128), lambda i: (i, 0, 0)),
        scratch_shapes=[
            pltpu.VMEM((_R2, 288), _BF),
            pltpu.VMEM((_R2, 64), _F32),
            pltpu.VMEM((_R3, 576), _BF),
            pltpu.VMEM((_R4, 1152), _BF),
            pltpu.VMEM((_R4, 128), _F32),
        ],
        compiler_params=pltpu.CompilerParams(
            dimension_semantics=("arbitrary",),
            vmem_limit_bytes=48 * 1024 * 1024),
    )(p1, w1b, w2b, w3b, w4b)


def _fc_kernel(x_ref, w1_ref, b1_ref, w3_ref, b3_ref, o_ref, h_ref):
    k = pl.program_id(0)

    @pl.when(k == 0)
    def _():
        h_ref[...] = jnp.zeros_like(h_ref)

    h_ref[...] += jnp.dot(x_ref[...], w1_ref[...],
                          preferred_element_type=_F32)

    @pl.when(k == pl.num_programs(0) - 1)
    def _():
        h = h_ref[...] + b1_ref[...]
        o_ref[...] = jnp.dot(h, w3_ref[...],
                             preferred_element_type=_F32) + b3_ref[...]


def _fc_head(feats, w1, b1, w3, b3):
    n, kdim = feats.shape
    nh = w1.shape[1]
    nc = w3.shape[1]
    tk = 3456
    return pl.pallas_call(
        _fc_kernel,
        out_shape=jax.ShapeDtypeStruct((n, nc), _F32),
        grid=(kdim // tk,),
        in_specs=[
            pl.BlockSpec((n, tk), lambda k: (0, k)),
            pl.BlockSpec((tk, nh), lambda k: (k, 0)),
            pl.BlockSpec((1, nh), lambda k: (0, 0)),
            pl.BlockSpec((nh, nc), lambda k: (0, 0)),
            pl.BlockSpec((1, nc), lambda k: (0, 0)),
        ],
        out_specs=pl.BlockSpec((n, nc), lambda k: (0, 0)),
        scratch_shapes=[pltpu.VMEM((n, nh), _F32)],
        compiler_params=pltpu.CompilerParams(
            dimension_semantics=("arbitrary",),
            vmem_limit_bytes=44 * 1024 * 1024),
    )(feats, w1, b1, w3, b3)


def _forward(w1p, w2r, w3r, w4r, fc1_w, fc1_b, fc3_w, fc3_b, x):
    n = x.shape[0]
    xf = jnp.pad(x.reshape(n, _H0 * _H0), ((0, 0), (0, 4))).astype(_BF)
    offs25 = [p * _H0 + q for p in range(5) for q in range(5)]
    p1 = jnp.stack([xf[:, o:o + _R2] for o in offs25], axis=-1)
    w1_33 = w1p[0:9, :].reshape(3, 3, 32)
    w1b = jnp.concatenate(
        [jnp.pad(w1_33, ((di, 2 - di), (dj, 2 - dj), (0, 0))).reshape(25, 32)
         for di in range(3) for dj in range(3)], axis=1).astype(_BF)
    w2b = w2r.reshape(288, 64).astype(_BF)
    w3b = w3r.reshape(576, 128).astype(_BF)
    w4b = w4r.reshape(1152, 128).astype(_BF)
    feats = _conv_stack(p1, w1b, w2b, w3b, w4b)
    return _fc_head(feats.reshape(n, 81 * 128), fc1_w, fc1_b, fc3_w, fc3_b)


def kernel(w1p, w2r, w3r, w4r, fc1_w, fc1_b, fc3_w, fc3_b, x):
    args = (w1p, w2r, w3r, w4r, fc1_w, fc1_b, fc3_w, fc3_b, x)
    devs = jax.devices()
    if len(devs) < 2 or x.shape[0] % (2 * _IPS) != 0:
        return _forward(*args)
    mesh = jax.sharding.Mesh(np.asarray(devs[:2]), ("b",))
    rep = jax.sharding.PartitionSpec()
    fwd = jax.experimental.shard_map.shard_map(
        _forward, mesh=mesh,
        in_specs=tuple([rep] * 8 + [jax.sharding.PartitionSpec("b")]),
        out_specs=jax.sharding.PartitionSpec("b"), check_rep=False)
    return fwd(*args)


# speedup vs baseline: 1.2794x; 1.2794x over previous
import jax
import jax.numpy as jnp
from jax.experimental import pallas as pl
from jax.experimental.pallas import tpu as pltpu

_H0 = 48
_R1 = 46 * 48
_R2 = 44 * 48
_WB = 24
_R3 = 20 * _WB
_R4 = 18 * _WB
_OFFS48 = tuple(di * 48 + dj for di in range(3) for dj in range(3))
_OFFS24 = tuple(di * _WB + dj for di in range(3) for dj in range(3))

_BF = jnp.bfloat16
_F32 = jnp.float32


_IPS = 4


def _conv_stack_kernel(p1_ref, w1_ref, w2_ref, w3_ref, w4_ref, o_ref,
                       acc2, p3, p4, acc4):
    for j in range(_IPS):
        c1 = jnp.dot(p1_ref[j], w1_ref[...], preferred_element_type=_F32)
        p2v = jnp.maximum(c1, 0.0).astype(_BF)
        acc2[...] = jnp.dot(p2v, w2_ref[...], preferred_element_type=_F32)

        jmax = jnp.maximum(acc2[pl.ds(0, 1056, 2), :],
                           acc2[pl.ds(1, 1056, 2), :]).reshape(22, 2, _WB, 64)
        cv2 = jnp.maximum(jnp.maximum(jmax[:, 0], jmax[:, 1]),
                          0.0).astype(_BF).reshape(528, 64)
        for t, off in enumerate(_OFFS24):
            sz = min(_R3, 528 - off)
            p3[0:sz, 64 * t:64 * (t + 1)] = cv2[off:off + sz, :]

        c3 = jnp.dot(p3[...], w3_ref[...], preferred_element_type=_F32)
        cv3 = jnp.maximum(c3, 0.0).astype(_BF)
        for t, off in enumerate(_OFFS24):
            sz = min(_R4, _R3 - off)
            p4[0:sz, 128 * t:128 * (t + 1)] = cv3[off:off + sz, :]

        acc4[...] = jnp.dot(p4[...], w4_ref[...], preferred_element_type=_F32)

        for ph in range(9):
            b0 = (2 * ph) * _WB
            b1 = b0 + _WB
            pr = jnp.maximum(
                jnp.maximum(acc4[pl.ds(b0, 12, 2), :],
                            acc4[pl.ds(b0 + 1, 12, 2), :]),
                jnp.maximum(acc4[pl.ds(b1, 12, 2), :],
                            acc4[pl.ds(b1 + 1, 12, 2), :]))
            pr = jnp.maximum(pr, 0.0)
            o_ref[j, pl.ds(ph * 9, 9), :] = pr[0:9, :]


def _conv_stack(p1, w1b, w2b, w3b, w4b):
    n = p1.shape[0]
    return pl.pallas_call(
        _conv_stack_kernel,
        out_shape=jax.ShapeDtypeStruct((n, 81, 128), _F32),
        grid=(n // _IPS,),
        in_specs=[
            pl.BlockSpec((_IPS, _R2, 25), lambda i: (i, 0, 0)),
            pl.BlockSpec((25, 288), lambda i: (0, 0)),
            pl.BlockSpec((288, 64), lambda i: (0, 0)),
            pl.BlockSpec((576, 128), lambda i: (0, 0)),
            pl.BlockSpec((1152, 128), lambda i: (0, 0)),
        ],
        out_specs=pl.BlockSpec((_IPS, 81, 128), lambda i: (i, 0, 0)),
        scratch_shapes=[
            pltpu.VMEM((_R2, 64), _F32),
            pltpu.VMEM((_R3, 576), _BF),
            pltpu.VMEM((_R4, 1152), _BF),
            pltpu.VMEM((_R4, 128), _F32),
        ],
        compiler_params=pltpu.CompilerParams(
            dimension_semantics=("arbitrary",),
            vmem_limit_bytes=48 * 1024 * 1024),
    )(p1, w1b, w2b, w3b, w4b)


def _fc_kernel(x_ref, w1_ref, b1_ref, w3_ref, b3_ref, o_ref, h_ref):
    k = pl.program_id(0)

    @pl.when(k == 0)
    def _():
        h_ref[...] = jnp.zeros_like(h_ref)

    h_ref[...] += jnp.dot(x_ref[...], w1_ref[...],
                          preferred_element_type=_F32)

    @pl.when(k == pl.num_programs(0) - 1)
    def _():
        h = h_ref[...] + b1_ref[...]
        o_ref[...] = jnp.dot(h, w3_ref[...],
                             preferred_element_type=_F32) + b3_ref[...]


def _fc_head(feats, w1, b1, w3, b3):
    n, kdim = feats.shape
    nh = w1.shape[1]
    nc = w3.shape[1]
    tk = 3456
    return pl.pallas_call(
        _fc_kernel,
        out_shape=jax.ShapeDtypeStruct((n, nc), _F32),
        grid=(kdim // tk,),
        in_specs=[
            pl.BlockSpec((n, tk), lambda k: (0, k)),
            pl.BlockSpec((tk, nh), lambda k: (k, 0)),
            pl.BlockSpec((1, nh), lambda k: (0, 0)),
            pl.BlockSpec((nh, nc), lambda k: (0, 0)),
            pl.BlockSpec((1, nc), lambda k: (0, 0)),
        ],
        out_specs=pl.BlockSpec((n, nc), lambda k: (0, 0)),
        scratch_shapes=[pltpu.VMEM((n, nh), _F32)],
        compiler_params=pltpu.CompilerParams(
            dimension_semantics=("arbitrary",),
            vmem_limit_bytes=44 * 1024 * 1024),
    )(feats, w1, b1, w3, b3)


def _forward(w1p, w2r, w3r, w4r, fc1_w, fc1_b, fc3_w, fc3_b, x):
    n = x.shape[0]
    xf = jnp.pad(x.reshape(n, _H0 * _H0), ((0, 0), (0, 4))).astype(_BF)
    offs25 = [p * _H0 + q for p in range(5) for q in range(5)]
    p1 = jnp.stack([xf[:, o:o + _R2] for o in offs25], axis=-1)
    w1_33 = w1p[0:9, :].reshape(3, 3, 32)
    w1b = jnp.concatenate(
        [jnp.pad(w1_33, ((di, 2 - di), (dj, 2 - dj), (0, 0))).reshape(25, 32)
         for di in range(3) for dj in range(3)], axis=1).astype(_BF)
    w2b = w2r.reshape(288, 64).astype(_BF)
    w3b = w3r.reshape(576, 128).astype(_BF)
    w4b = w4r.reshape(1152, 128).astype(_BF)
    feats = _conv_stack(p1, w1b, w2b, w3b, w4b)
    return _fc_head(feats.reshape(n, 81 * 128), fc1_w, fc1_b, fc3_w, fc3_b)


def kernel(w1p, w2r, w3r, w4r, fc1_w, fc1_b, fc3_w, fc3_b, x):
    return _forward(w1p, w2r, w3r, w4r, fc1_w, fc1_b, fc3_w, fc3_b, x)


# speedup vs baseline: 1.3166x; 1.0291x over previous
import jax
import jax.numpy as jnp
from jax.experimental import pallas as pl
from jax.experimental.pallas import tpu as pltpu

_H0 = 48
_R1 = 46 * 48
_R2 = 44 * 48
_WB = 24
_R3 = 20 * _WB
_R4 = 18 * _WB
_OFFS48 = tuple(di * 48 + dj for di in range(3) for dj in range(3))
_OFFS24 = tuple(di * _WB + dj for di in range(3) for dj in range(3))

_BF = jnp.bfloat16
_F32 = jnp.float32


_IPS = 8


def _conv_stack_kernel(p1_ref, w1_ref, w2_ref, w3_ref, w4_ref, o_ref,
                       acc2, p3, p4, acc4):
    for j in range(_IPS):
        c1 = jnp.dot(p1_ref[j], w1_ref[...], preferred_element_type=_F32)
        p2v = jnp.maximum(c1, 0.0).astype(_BF)
        acc2[...] = jnp.dot(p2v, w2_ref[...], preferred_element_type=_F32)

        jmax = jnp.maximum(acc2[pl.ds(0, 1056, 2), :],
                           acc2[pl.ds(1, 1056, 2), :]).reshape(22, 2, _WB, 64)
        cv2 = jnp.maximum(jnp.maximum(jmax[:, 0], jmax[:, 1]),
                          0.0).astype(_BF).reshape(528, 64)
        for t, off in enumerate(_OFFS24):
            sz = min(_R3, 528 - off)
            p3[0:sz, 64 * t:64 * (t + 1)] = cv2[off:off + sz, :]

        c3 = jnp.dot(p3[...], w3_ref[...], preferred_element_type=_F32)
        cv3 = jnp.maximum(c3, 0.0).astype(_BF)
        for t, off in enumerate(_OFFS24):
            sz = min(_R4, _R3 - off)
            p4[0:sz, 128 * t:128 * (t + 1)] = cv3[off:off + sz, :]

        acc4[...] = jnp.dot(p4[...], w4_ref[...], preferred_element_type=_F32)

        for ph in range(9):
            b0 = (2 * ph) * _WB
            b1 = b0 + _WB
            pr = jnp.maximum(
                jnp.maximum(acc4[pl.ds(b0, 12, 2), :],
                            acc4[pl.ds(b0 + 1, 12, 2), :]),
                jnp.maximum(acc4[pl.ds(b1, 12, 2), :],
                            acc4[pl.ds(b1 + 1, 12, 2), :]))
            pr = jnp.maximum(pr, 0.0)
            o_ref[j, pl.ds(ph * 9, 9), :] = pr[0:9, :]


def _conv_stack(p1, w1b, w2b, w3b, w4b):
    n = p1.shape[0]
    return pl.pallas_call(
        _conv_stack_kernel,
        out_shape=jax.ShapeDtypeStruct((n, 81, 128), _F32),
        grid=(n // _IPS,),
        in_specs=[
            pl.BlockSpec((_IPS, _R2, 25), lambda i: (i, 0, 0)),
            pl.BlockSpec((25, 288), lambda i: (0, 0)),
            pl.BlockSpec((288, 64), lambda i: (0, 0)),
            pl.BlockSpec((576, 128), lambda i: (0, 0)),
            pl.BlockSpec((1152, 128), lambda i: (0, 0)),
        ],
        out_specs=pl.BlockSpec((_IPS, 81, 128), lambda i: (i, 0, 0)),
        scratch_shapes=[
            pltpu.VMEM((_R2, 64), _F32),
            pltpu.VMEM((_R3, 576), _BF),
            pltpu.VMEM((_R4, 1152), _BF),
            pltpu.VMEM((_R4, 128), _F32),
        ],
        compiler_params=pltpu.CompilerParams(
            dimension_semantics=("arbitrary",),
            vmem_limit_bytes=48 * 1024 * 1024),
    )(p1, w1b, w2b, w3b, w4b)


def _fc_kernel(x_ref, w1_ref, b1_ref, w3_ref, b3_ref, o_ref, h_ref):
    k = pl.program_id(0)

    @pl.when(k == 0)
    def _():
        h_ref[...] = jnp.zeros_like(h_ref)

    h_ref[...] += jnp.dot(x_ref[...], w1_ref[...],
                          preferred_element_type=_F32)

    @pl.when(k == pl.num_programs(0) - 1)
    def _():
        h = h_ref[...] + b1_ref[...]
        o_ref[...] = jnp.dot(h, w3_ref[...],
                             preferred_element_type=_F32) + b3_ref[...]


def _fc_head(feats, w1, b1, w3, b3):
    n, kdim = feats.shape
    nh = w1.shape[1]
    nc = w3.shape[1]
    tk = 3456
    return pl.pallas_call(
        _fc_kernel,
        out_shape=jax.ShapeDtypeStruct((n, nc), _F32),
        grid=(kdim // tk,),
        in_specs=[
            pl.BlockSpec((n, tk), lambda k: (0, k)),
            pl.BlockSpec((tk, nh), lambda k: (k, 0)),
            pl.BlockSpec((1, nh), lambda k: (0, 0)),
            pl.BlockSpec((nh, nc), lambda k: (0, 0)),
            pl.BlockSpec((1, nc), lambda k: (0, 0)),
        ],
        out_specs=pl.BlockSpec((n, nc), lambda k: (0, 0)),
        scratch_shapes=[pltpu.VMEM((n, nh), _F32)],
        compiler_params=pltpu.CompilerParams(
            dimension_semantics=("arbitrary",),
            vmem_limit_bytes=44 * 1024 * 1024),
    )(feats, w1, b1, w3, b3)


def _forward(w1p, w2r, w3r, w4r, fc1_w, fc1_b, fc3_w, fc3_b, x):
    n = x.shape[0]
    xf = jnp.pad(x.reshape(n, _H0 * _H0), ((0, 0), (0, 4))).astype(_BF)
    offs25 = [p * _H0 + q for p in range(5) for q in range(5)]
    p1 = jnp.stack([xf[:, o:o + _R2] for o in offs25], axis=-1)
    w1_33 = w1p[0:9, :].reshape(3, 3, 32)
    w1b = jnp.concatenate(
        [jnp.pad(w1_33, ((di, 2 - di), (dj, 2 - dj), (0, 0))).reshape(25, 32)
         for di in range(3) for dj in range(3)], axis=1).astype(_BF)
    w2b = w2r.reshape(288, 64).astype(_BF)
    w3b = w3r.reshape(576, 128).astype(_BF)
    w4b = w4r.reshape(1152, 128).astype(_BF)
    feats = _conv_stack(p1, w1b, w2b, w3b, w4b)
    return _fc_head(feats.reshape(n, 81 * 128), fc1_w, fc1_b, fc3_w, fc3_b)


def kernel(w1p, w2r, w3r, w4r, fc1_w, fc1_b, fc3_w, fc3_b, x):
    return _forward(w1p, w2r, w3r, w4r, fc1_w, fc1_b, fc3_w, fc3_b, x)
